# Initial kernel scaffold; baseline (speedup 1.0000x reference)
import jax
import jax.numpy as jnp
from jax.experimental import pallas as pl


def kernel(hs, index_list):
    raise NotImplementedError("write your pallas kernel here")


# speedup vs baseline: 9.8188x; 9.8188x over previous
import functools

import jax
import jax.numpy as jnp
from jax import lax
from jax.experimental import pallas as pl
from jax.experimental.pallas import tpu as pltpu
from jax.experimental.pallas import tpu_sc as plsc

_N = 316
_N2 = _N * _N
_NC = 2
_NS = 16
_NW = _NC * _NS
_CH = 3128
_HALO = 640
_HSLICE = _CH + 2 * _HALO
_JTOT = 13 * _CH
_STEPS = (_JTOT + 15) // 16
_OUTW = 26 * _CH

_mesh = plsc.VectorSubcoreMesh(
    core_axis_name="c", subcore_axis_name="s", num_cores=_NC, num_subcores=_NS
)


@functools.partial(
    pl.kernel,
    mesh=_mesh,
    out_type=jax.ShapeDtypeStruct((_N2 * 26,), jnp.float32),
    scratch_types=[
        pltpu.VMEM((_HSLICE,), jnp.float32),
        pltpu.VMEM((_STEPS * 16,), jnp.int32),
        pltpu.VMEM((_OUTW,), jnp.float32),
    ],
    compiler_params=pltpu.CompilerParams(needs_layout_passes=False),
)
def _sc_stencil(hs_hbm, idx_hbm, out_hbm, hs_v, idx_v, out_v):
    wid = lax.axis_index("s") * _NC + lax.axis_index("c")
    row0 = pl.multiple_of(jnp.minimum(wid * _CH, _N2 - _CH), 8)
    lo = pl.multiple_of(jnp.clip(row0 - _HALO, 0, _N2 - _HSLICE), 8)
    pltpu.sync_copy(hs_hbm.at[pl.ds(lo, _HSLICE)], hs_v)
    pltpu.sync_copy(
        idx_hbm.at[pl.ds(pl.multiple_of(row0 * 13, 8), _JTOT)],
        idx_v.at[pl.ds(0, _JTOT)],
    )

    roff = row0 - lo
    iota = lax.iota(jnp.int32, 16)
    r0 = jnp.where(iota < 13, 0, 1)
    k0 = jnp.where(iota < 13, iota, iota - 13)

    def step(s, carry):
        r, k = carry
        jbase = s * 16
        m = (iota + jbase) < _JTOT
        iv = idx_v[pl.ds(jbase, 16)]
        g = plsc.load_gather(hs_v, [iv - lo], mask=m)
        sv = plsc.load_gather(hs_v, [r + roff], mask=m)
        dst = r * 26 + k * 2
        plsc.store_scatter(out_v, [dst], sv, mask=m)
        plsc.store_scatter(out_v, [dst + 1], g, mask=m)
        k2 = k + 3
        wrap = k2 >= 13
        k2 = jnp.where(wrap, k2 - 13, k2)
        r2 = r + jnp.where(wrap, 2, 1)
        return r2, k2

    lax.fori_loop(0, _STEPS, step, (r0, k0))
    pltpu.sync_copy(out_v, out_hbm.at[pl.ds(pl.multiple_of(row0 * 26, 8), _OUTW)])


def kernel(hs, index_list):
    out_flat = _sc_stencil(hs, index_list)
    return out_flat.reshape(_N2, 13, 2)


# speedup vs baseline: 10.0128x; 1.0198x over previous
import functools

import jax
import jax.numpy as jnp
from jax import lax
from jax.experimental import pallas as pl
from jax.experimental.pallas import tpu as pltpu
from jax.experimental.pallas import tpu_sc as plsc

_N = 316
_N2 = _N * _N
_NC = 2
_NS = 16
_NW = _NC * _NS
_CH = 3128
_HALO = 640
_HSLICE = _CH + 2 * _HALO
_JTOT = 13 * _CH
_STEPS = (_JTOT + 15) // 16
_OUTW = 26 * _CH

_mesh = plsc.VectorSubcoreMesh(
    core_axis_name="c", subcore_axis_name="s", num_cores=_NC, num_subcores=_NS
)


@functools.partial(
    pl.kernel,
    mesh=_mesh,
    out_type=jax.ShapeDtypeStruct((_N2 * 26,), jnp.float32),
    scratch_types=[
        pltpu.VMEM((_HSLICE,), jnp.float32),
        pltpu.VMEM((_STEPS * 16,), jnp.int32),
        pltpu.VMEM((_OUTW,), jnp.float32),
    ],
    compiler_params=pltpu.CompilerParams(needs_layout_passes=False),
)
def _sc_stencil(hs_hbm, idx_hbm, out_hbm, hs_v, idx_v, out_v):
    wid = lax.axis_index("s") * _NC + lax.axis_index("c")
    row0 = pl.multiple_of(jnp.minimum(wid * _CH, _N2 - _CH), 8)
    lo = pl.multiple_of(jnp.clip(row0 - _HALO, 0, _N2 - _HSLICE), 8)
    pltpu.sync_copy(hs_hbm.at[pl.ds(lo, _HSLICE)], hs_v)
    pltpu.sync_copy(
        idx_hbm.at[pl.ds(pl.multiple_of(row0 * 13, 8), _JTOT)],
        idx_v.at[pl.ds(0, _JTOT)],
    )

    roff = row0 - lo
    iota = lax.iota(jnp.int32, 16)
    r0 = jnp.where(iota < 13, 0, 1)
    k0 = jnp.where(iota < 13, iota, iota - 13)

    @plsc.parallel_loop(0, _STEPS - 1, unroll=13, carry=(r0, k0))
    def _main(s, carry):
        r, k = carry
        iv = idx_v[pl.ds(s * 16, 16)]
        g = plsc.load_gather(hs_v, [iv - lo])
        sv = plsc.load_gather(hs_v, [r + roff])
        dst = r * 26 + k * 2
        plsc.store_scatter(out_v, [dst], sv)
        plsc.store_scatter(out_v, [dst + 1], g)
        k2 = k + 3
        wrap = k2 >= 13
        k2 = jnp.where(wrap, k2 - 13, k2)
        r2 = r + jnp.where(wrap, 2, 1)
        return r2, k2

    r, k = _main
    jbase = (_STEPS - 1) * 16
    m = (iota + jbase) < _JTOT
    iv = idx_v[pl.ds(jbase, 16)]
    g = plsc.load_gather(hs_v, [jnp.where(m, iv - lo, 0)], mask=m)
    sv = plsc.load_gather(hs_v, [r + roff], mask=m)
    dst = r * 26 + k * 2
    plsc.store_scatter(out_v, [dst], sv, mask=m)
    plsc.store_scatter(out_v, [dst + 1], g, mask=m)
    pltpu.sync_copy(out_v, out_hbm.at[pl.ds(pl.multiple_of(row0 * 26, 8), _OUTW)])


def kernel(hs, index_list):
    out_flat = _sc_stencil(hs, index_list)
    return out_flat.reshape(_N2, 13, 2)


# speedup vs baseline: 140.7232x; 14.0544x over previous
import jax
import jax.numpy as jnp
from jax import lax
from jax.experimental import pallas as pl
from jax.experimental.pallas import tpu as pltpu
from jax.experimental.pallas import tpu_sc as plsc

_N = 316
_N2 = _N * _N
_NC = 2
_NS = 16
_CH = 3128
_HALO = 640
_HSLICE = _CH + 2 * _HALO
_JTOT = 13 * _CH
_STEPS = (_JTOT + 15) // 16

_mesh = plsc.VectorSubcoreMesh(
    core_axis_name="c", subcore_axis_name="s", num_cores=_NC, num_subcores=_NS
)


@pl.kernel(
    mesh=_mesh,
    out_type=jax.ShapeDtypeStruct((13 * _N2,), jnp.float32),
    scratch_types=[
        pltpu.VMEM((_HSLICE,), jnp.float32),
        pltpu.VMEM((_STEPS * 16,), jnp.int32),
        pltpu.VMEM((13 * _CH,), jnp.float32),
        pltpu.SemaphoreType.DMA,
    ],
    compiler_params=pltpu.CompilerParams(needs_layout_passes=False),
)
def _sc_gather(hs_hbm, idx_hbm, out_hbm, hs_v, idx_v, out_v, sem):
    wid = lax.axis_index("s") * _NC + lax.axis_index("c")
    row0 = pl.multiple_of(jnp.minimum(wid * _CH, _N2 - _CH), 8)
    lo = pl.multiple_of(jnp.clip(row0 - _HALO, 0, _N2 - _HSLICE), 8)
    pltpu.sync_copy(hs_hbm.at[pl.ds(lo, _HSLICE)], hs_v)
    pltpu.sync_copy(
        idx_hbm.at[pl.ds(pl.multiple_of(row0 * 13, 8), _JTOT)],
        idx_v.at[pl.ds(0, _JTOT)],
    )

    iota = lax.iota(jnp.int32, 16)
    r0 = jnp.where(iota < 13, 0, 1)
    k0 = jnp.where(iota < 13, iota, iota - 13)

    @plsc.parallel_loop(0, _STEPS - 1, unroll=13, carry=(r0, k0))
    def _main(s, carry):
        r, k = carry
        iv = idx_v[pl.ds(s * 16, 16)]
        g = plsc.load_gather(hs_v, [iv - lo])
        plsc.store_scatter(out_v, [k * _CH + r], g)
        k2 = k + 3
        wrap = k2 >= 13
        k2 = jnp.where(wrap, k2 - 13, k2)
        r2 = r + jnp.where(wrap, 2, 1)
        return r2, k2

    r, k = _main
    jbase = (_STEPS - 1) * 16
    m = (iota + jbase) < _JTOT
    iv = idx_v[pl.ds(jbase, 16)]
    g = plsc.load_gather(hs_v, [jnp.where(m, iv - lo, 0)], mask=m)
    plsc.store_scatter(out_v, [k * _CH + r], g, mask=m)

    copies = [
        pltpu.async_copy(
            out_v.at[pl.ds(k * _CH, _CH)],
            out_hbm.at[pl.ds(pl.multiple_of(k * _N2 + row0, 8), _CH)],
            sem,
        )
        for k in range(13)
    ]
    for c in copies:
        c.wait()


def kernel(hs, index_list):
    neigh = _sc_gather(hs, index_list).reshape(13, _N2)
    self_part = jnp.broadcast_to(hs.reshape(_N2, 1), (_N2, 13))
    return jnp.stack([self_part, neigh.T], axis=-1)


# speedup vs baseline: 144.5376x; 1.0271x over previous
import jax
import jax.numpy as jnp
from jax import lax
from jax.experimental import pallas as pl
from jax.experimental.pallas import tpu as pltpu
from jax.experimental.pallas import tpu_sc as plsc

_N = 316
_N2 = _N * _N
_NC = 2
_NS = 16
_NT = 781
_CH = 3200
_HALO = 640
_HSLICE = _CH + 2 * _HALO
_JTOT = 13 * _CH
_STEPS = _JTOT // 16
_WCHUNK = 25 * 256
_OUTW = 13 * _WCHUNK
_PLANE = _NT * 256

_mesh = plsc.VectorSubcoreMesh(
    core_axis_name="c", subcore_axis_name="s", num_cores=_NC, num_subcores=_NS
)


@pl.kernel(
    mesh=_mesh,
    out_type=jax.ShapeDtypeStruct((13 * _NT * 2 * 128,), jnp.float32),
    scratch_types=[
        pltpu.VMEM((_HSLICE,), jnp.float32),
        pltpu.VMEM((_JTOT,), jnp.int32),
        pltpu.VMEM((_OUTW,), jnp.float32),
        pltpu.SemaphoreType.DMA,
    ],
    compiler_params=pltpu.CompilerParams(needs_layout_passes=False),
)
def _sc_stencil(hs_hbm, idx_hbm, out_hbm, hs_v, idx_v, out_v, sem):
    wid = lax.axis_index("s") * _NC + lax.axis_index("c")
    row0 = pl.multiple_of(jnp.minimum(wid * _CH, 96768), 128)
    jstart = pl.multiple_of(jnp.minimum(row0 * 13, 13 * _N2 - _JTOT), 8)
    roff2 = row0 - jstart // 13
    lo = pl.multiple_of(jnp.clip(row0 - _HALO, 0, _N2 - _HSLICE), 8)
    pltpu.sync_copy(hs_hbm.at[pl.ds(lo, _HSLICE)], hs_v)
    pltpu.sync_copy(idx_hbm.at[pl.ds(jstart, _JTOT)], idx_v)

    soff = row0 - lo
    iota = lax.iota(jnp.int32, 16)
    jr0 = jnp.where(iota < 13, 0, 1)
    k0 = jnp.where(iota < 13, iota, iota - 13) * _WCHUNK

    @plsc.parallel_loop(0, _STEPS, unroll=13, carry=(jr0, k0))
    def _main(s, carry):
        jr, k64 = carry
        iv = idx_v[pl.ds(s * 16, 16)]
        g = plsc.load_gather(hs_v, [iv - lo])
        r = jr - roff2
        m = r >= 0
        sv = plsc.load_gather(hs_v, [r + soff])
        dst = k64 + r + (r & -128)
        plsc.store_scatter(out_v, [dst], sv, mask=m)
        plsc.store_scatter(out_v, [dst + 128], g, mask=m)
        k2 = k64 + 3 * _WCHUNK
        wrap = k2 >= _OUTW
        k64n = jnp.where(wrap, k2 - _OUTW, k2)
        jrn = jr + jnp.where(wrap, 2, 1)
        return jrn, k64n

    del _main
    copies = [
        pltpu.async_copy(
            out_v.at[pl.ds(k * _WCHUNK, _WCHUNK)],
            out_hbm.at[pl.ds(pl.multiple_of(k * _PLANE + 2 * row0, 8), _WCHUNK)],
            sem,
        )
        for k in range(13)
    ]
    for c in copies:
        c.wait()


def kernel(hs, index_list):
    flat = _sc_stencil(hs, index_list)
    x = flat.reshape(13, _NT, 2, 128)
    y = x.transpose(1, 3, 0, 2).reshape(_NT * 128, 13, 2)
    return y[:_N2]


# speedup vs baseline: 208.3268x; 1.4413x over previous
import jax
import jax.numpy as jnp
from jax import lax
from jax.experimental import pallas as pl
from jax.experimental.pallas import tpu as pltpu
from jax.experimental.pallas import tpu_sc as plsc

_N = 316
_N2 = _N * _N
_NC = 2
_NS = 16
_NT = 781
_CH = 3200
_TPW = 25
_HALO = 640
_HSLICE = _CH + 2 * _HALO
_JTOT = 13 * _CH

_mesh = plsc.VectorSubcoreMesh(
    core_axis_name="c", subcore_axis_name="s", num_cores=_NC, num_subcores=_NS
)


@pl.kernel(
    mesh=_mesh,
    out_type=jax.ShapeDtypeStruct((13, _NT, 2, 128), jnp.float32),
    scratch_types=[
        pltpu.VMEM((_HSLICE,), jnp.float32),
        pltpu.VMEM((_JTOT,), jnp.int32),
        pltpu.VMEM((13, _TPW, 2, 128), jnp.float32),
        pltpu.SemaphoreType.DMA,
        pltpu.SemaphoreType.DMA,
    ],
    compiler_params=pltpu.CompilerParams(needs_layout_passes=False),
)
def _sc_stencil(hs_hbm, idx_hbm, out_hbm, hs_v, idx_v, out_v, sem_in, sem_out):
    wid = lax.axis_index("s") * _NC + lax.axis_index("c")
    row0 = pl.multiple_of(jnp.minimum(wid * _CH, 96768), 128)
    it0 = jnp.minimum(wid * _TPW, _NT - _TPW)
    jstart = pl.multiple_of(jnp.minimum(row0 * 13, 13 * _N2 - _JTOT), 8)
    roff = row0 - jstart // 13
    lo = pl.multiple_of(jnp.clip(row0 - _HALO, 0, _N2 - _HSLICE), 8)
    idx_dma = pltpu.async_copy(idx_hbm.at[pl.ds(jstart, _JTOT)], idx_v, sem_in)
    pltpu.sync_copy(hs_hbm.at[pl.ds(lo, _HSLICE)], hs_v)

    soff = row0 - lo
    send = _HSLICE - 16 - soff
    iota13 = lax.iota(jnp.int32, 16) * 13

    @plsc.parallel_loop(0, _TPW, unroll=5)
    def _selfpass(it):
        for b in range(8):
            s0 = jnp.minimum(it * 128 + b * 16, send)
            v = hs_v[pl.ds(s0 + soff, 16)]
            for k in range(13):
                out_v[k, it, 0, pl.ds(b * 16, 16)] = v

    del _selfpass
    idx_dma.wait()

    out_copies = []
    for k in range(13):
        bk = roff * 13 + k

        @plsc.parallel_loop(0, _TPW, unroll=5)
        def _kpass(it, _bk=bk, _k=k):
            base = it * (128 * 13) + _bk
            for b in range(8):
                pos = jnp.minimum(base + b * (16 * 13) + iota13, _JTOT - 1)
                iv = plsc.load_gather(idx_v, [pos])
                g = plsc.load_gather(hs_v, [jnp.clip(iv - lo, 0, _HSLICE - 1)])
                out_v[_k, it, 1, pl.ds(b * 16, 16)] = g

        del _kpass
        out_copies.append(
            pltpu.async_copy(out_v.at[k], out_hbm.at[k, pl.ds(it0, _TPW)], sem_out)
        )
    for c in out_copies:
        c.wait()


def kernel(hs, index_list):
    x = _sc_stencil(hs, index_list)
    y = x.transpose(1, 3, 0, 2).reshape(_NT * 128, 13, 2)
    return y[:_N2]
